# baseline (device time: 277163 ns/iter reference)
import jax
import jax.numpy as jnp
from jax import lax
from jax.experimental import pallas as pl
from jax.experimental.pallas import tpu as pltpu

N_DEV = 16
B, S, H, Dh, Dr = 4, 256, 32, 128, 64
D = 4096
HB = D // N_DEV
NSLOTS = 2
NSTEP = N_DEV - 1
MESH = pl.DeviceIdType.MESH


class _Chain:

    def __init__(self, tgt, peer, src_slice, dst_slice, recv, ssem, rsem,
                 credit, add_slice=None):
        self.tgt = tgt
        self.peer = peer
        self.src_slice = src_slice
        self.dst_slice = dst_slice
        self.add_slice = add_slice
        self.recv = recv
        self.ssem = ssem
        self.rsem = rsem
        self.credit = credit
        self.cur = None

    def make(self, s):
        slot = s % NSLOTS
        return pltpu.make_async_remote_copy(
            src_ref=self.src_slice(s), dst_ref=self.dst_slice(s),
            send_sem=self.ssem.at[slot], recv_sem=self.rsem.at[slot],
            device_id=(self.tgt,), device_id_type=MESH)

    def start(self, s):
        if s >= NSLOTS:
            pl.semaphore_wait(self.credit, 1)
        self.cur = self.make(s)
        self.cur.start()

    def finish(self, s):
        self.cur.wait()
        if self.add_slice is not None:
            slot = s % NSLOTS
            tgt = self.add_slice(s)
            tgt[...] = tgt[...] + self.recv[slot]
        if s < NSTEP - NSLOTS:
            pl.semaphore_signal(self.credit, inc=1, device_id=(self.peer,),
                                device_id_type=MESH)


def _run_chains(chains):
    for c in chains:
        c.start(0)
    for s in range(NSTEP):
        for c in chains:
            c.finish(s)
            if s + 1 < NSTEP:
                c.start(s + 1)


def _ring_rs_kv(k, v):
    CW = HB // 2

    def body(k_hbm, v_hbm, ko_ref, vo_ref, kbuf, vbuf,
             krecv_a, krecv_b, vrecv_a, vrecv_b,
             ks_a, kr_a, ks_b, kr_b, vs_a, vr_a, vs_b, vr_b,
             kcred_a, kcred_b, vcred_a, vcred_b, kcopy, vcopy):
        p = lax.axis_index("i")
        left = (p - 1) % N_DEV
        right = (p + 1) % N_DEV

        cpk = pltpu.make_async_copy(k_hbm, kbuf, kcopy)
        cpv = pltpu.make_async_copy(v_hbm, vbuf, vcopy)
        cpk.start()
        cpv.start()

        barrier_sem = pltpu.get_barrier_semaphore()
        for nbr in (left, right):
            pl.semaphore_signal(barrier_sem, inc=1, device_id=(nbr,),
                                device_id_type=MESH)
        pl.semaphore_wait(barrier_sem, 2)
        cpk.wait()
        cpv.wait()

        def strip(buf, idx, off):
            return buf.at[:, :, pl.ds(idx * HB + off, CW)]

        def mk_chain(buf, tgt, peer, send_idx, recv_idx, off, recv,
                     ssem, rsem, credit):
            return _Chain(
                tgt, peer,
                src_slice=lambda s: strip(buf, send_idx(s), off),
                dst_slice=lambda s: recv.at[s % NSLOTS],
                add_slice=lambda s: strip(buf, recv_idx(s), off),
                recv=recv, ssem=ssem, rsem=rsem, credit=credit)

        k_send = lambda s: (p - s) % N_DEV
        k_recv = lambda s: (p - s - 1) % N_DEV
        v_send = lambda s: (p + s + 2) % N_DEV
        v_recv = lambda s: (p + s + 3) % N_DEV

        chains = [
            mk_chain(kbuf, right, left, k_send, k_recv, 0, krecv_a,
                     ks_a, kr_a, kcred_a),
            mk_chain(vbuf, left, right, v_send, v_recv, 0, vrecv_a,
                     vs_a, vr_a, vcred_a),
            mk_chain(kbuf, right, left, k_send, k_recv, CW, krecv_b,
                     ks_b, kr_b, kcred_b),
            mk_chain(vbuf, left, right, v_send, v_recv, CW, vrecv_b,
                     vs_b, vr_b, vcred_b),
        ]
        _run_chains(chains)

        own = (p + 1) % N_DEV
        ko_ref[...] = kbuf[:, :, pl.ds(own * HB, HB)]
        vo_ref[...] = vbuf[:, :, pl.ds(own * HB, HB)]

    dma2 = pltpu.SemaphoreType.DMA((NSLOTS,))
    return pl.pallas_call(
        body,
        out_shape=[jax.ShapeDtypeStruct((B, S, HB), k.dtype),
                   jax.ShapeDtypeStruct((B, S, HB), k.dtype)],
        in_specs=[pl.BlockSpec(memory_space=pltpu.MemorySpace.HBM)] * 2,
        out_specs=[pl.BlockSpec(memory_space=pltpu.VMEM)] * 2,
        scratch_shapes=(
            [pltpu.VMEM((B, S, D), k.dtype)] * 2
            + [pltpu.VMEM((NSLOTS, B, S, CW), k.dtype)] * 4
            + [dma2] * 8
            + [pltpu.SemaphoreType.REGULAR] * 4
            + [pltpu.SemaphoreType.DMA] * 2
        ),
        compiler_params=pltpu.CompilerParams(
            collective_id=0, vmem_limit_bytes=63 * 1024 * 1024),
    )(k, v)


N_SUB = 16


def _ring_ar_out(y):
    SC = S // N_DEV
    Q = D // N_SUB

    def body(y_hbm, out_ref, wbuf, *scr):
        recvs = scr[0:N_SUB]
        rs_s = scr[N_SUB:2 * N_SUB]
        rs_r = scr[2 * N_SUB:3 * N_SUB]
        ag_s = scr[3 * N_SUB:4 * N_SUB]
        ag_r = scr[4 * N_SUB:5 * N_SUB]
        rcred = scr[5 * N_SUB:6 * N_SUB]
        acred = scr[6 * N_SUB:7 * N_SUB]
        copy_sem = scr[7 * N_SUB]

        p = lax.axis_index("i")
        left = (p - 1) % N_DEV
        right = (p + 1) % N_DEV

        cp = pltpu.make_async_copy(y_hbm, wbuf, copy_sem)
        cp.start()

        barrier_sem = pltpu.get_barrier_semaphore()
        for nbr in (left, right):
            pl.semaphore_signal(barrier_sem, inc=1, device_id=(nbr,),
                                device_id_type=MESH)
        pl.semaphore_wait(barrier_sem, 2)
        cp.wait()

        def strip(idx, q):
            return wbuf.at[:, pl.ds(idx * SC, SC), q * Q:(q + 1) * Q]

        f_rs_send = lambda s: (p - s) % N_DEV
        f_rs_recv = lambda s: (p - s - 1) % N_DEV
        r_rs_send = lambda s: (p + s) % N_DEV
        r_rs_recv = lambda s: (p + s + 1) % N_DEV
        f_ag_send = lambda s: (p + 1 - s) % N_DEV
        r_ag_send = lambda s: (p - 1 + s) % N_DEV

        def rs_chain(q):
            fwd = q < N_SUB // 2
            return _Chain(
                right if fwd else left, left if fwd else right,
                src_slice=(lambda s, q=q, f=fwd:
                           strip((f_rs_send if f else r_rs_send)(s), q)),
                dst_slice=lambda s, q=q: recvs[q].at[s % NSLOTS],
                add_slice=(lambda s, q=q, f=fwd:
                           strip((f_rs_recv if f else r_rs_recv)(s), q)),
                recv=recvs[q], ssem=rs_s[q], rsem=rs_r[q], credit=rcred[q])

        def ag_chain(q):
            fwd = q < N_SUB // 2
            send = f_ag_send if fwd else r_ag_send
            return _Chain(
                right if fwd else left, left if fwd else right,
                src_slice=lambda s, q=q, send=send: strip(send(s), q),
                dst_slice=lambda s, q=q, send=send: strip(send(s), q),
                add_slice=None,
                recv=None, ssem=ag_s[q], rsem=ag_r[q], credit=acred[q])

        half = N_SUB // 2
        order = [q for i in range(half) for q in (i, half + i)]
        _run_chains([rs_chain(q) for q in order])
        _run_chains([ag_chain(q) for q in order])

        out_ref[...] = wbuf[...].astype(jnp.float32)

    dma2 = pltpu.SemaphoreType.DMA((NSLOTS,))
    return pl.pallas_call(
        body,
        out_shape=jax.ShapeDtypeStruct((B, S, D), jnp.float32),
        in_specs=[pl.BlockSpec(memory_space=pltpu.MemorySpace.HBM)],
        out_specs=pl.BlockSpec(memory_space=pltpu.VMEM),
        scratch_shapes=(
            [pltpu.VMEM((B, S, D), y.dtype)]
            + [pltpu.VMEM((NSLOTS, B, SC, Q), y.dtype)] * N_SUB
            + [dma2] * (4 * N_SUB)
            + [pltpu.SemaphoreType.REGULAR] * (2 * N_SUB)
            + [pltpu.SemaphoreType.DMA]
        ),
        compiler_params=pltpu.CompilerParams(
            collective_id=1, vmem_limit_bytes=63 * 1024 * 1024),
    )(y)


def kernel(x, Wdkv, Wuk, Wuv, Wq, Wqr, Wkr, Wo):
    c = x @ Wdkv
    Kp = jnp.matmul(c, Wuk, preferred_element_type=jnp.bfloat16)
    Vp = jnp.matmul(c, Wuv, preferred_element_type=jnp.bfloat16)
    k_own, v_own = _ring_rs_kv(Kp, Vp)
    k_own = k_own.astype(jnp.float32)
    v_own = v_own.astype(jnp.float32)

    p = lax.axis_index("i")
    o = (p + 1) % N_DEV
    nh = HB // Dh

    Ko = k_own.reshape(B, S, nh, Dh)
    Vo = v_own.reshape(B, S, nh, Dh)
    Wq_o = lax.dynamic_slice(Wq, (0, o * HB), (D, HB))
    Wqr_o = lax.dynamic_slice(Wqr, (0, o * nh * Dr), (D, nh * Dr))
    Qo = (x @ Wq_o).reshape(B, S, nh, Dh)
    Qro = (x @ Wqr_o).reshape(B, S, nh, Dr)
    Kr = (x @ Wkr).reshape(B, S, 1, Dr)

    scale = (Dh + Dr) ** -0.5
    scores = (jnp.einsum("bshd,bthd->bhst", Qo, Ko)
              + jnp.einsum("bshd,bthd->bhst", Qro,
                           jnp.broadcast_to(Kr, (B, S, nh, Dr)))) * scale
    m = scores.max(-1, keepdims=True)
    P = jnp.exp(scores - m)
    P = P / P.sum(-1, keepdims=True)
    O = jnp.einsum("bhst,bthd->bshd", P, Vo).reshape(B, S, HB)

    Wo_o = lax.dynamic_slice(Wo, (o * HB, 0), (HB, D))
    y = jnp.matmul(O, Wo_o, preferred_element_type=jnp.bfloat16)
    return _ring_ar_out(y).astype(jnp.float32)
